# baseline (speedup 1.0000x reference)
import functools

import jax
import jax.numpy as jnp
from jax import lax
from jax.experimental import pallas as pl
from jax.experimental.pallas import tpu as pltpu
from jax.experimental.pallas import tpu_sc as plsc

BATCH = 16384
D = 64
L = 16
NW = 32
BPW = BATCH // NW
HB = BPW // 2
NG = BPW // L
HG = NG // 2

_mesh = plsc.VectorSubcoreMesh(core_axis_name="c", subcore_axis_name="s")


@functools.partial(
    pl.kernel,
    mesh=_mesh,
    out_type=(
        jax.ShapeDtypeStruct((BATCH,), jnp.float32),
        jax.ShapeDtypeStruct((BATCH,), jnp.float32),
    ),
    scratch_types=[
        pltpu.VMEM((BPW,), jnp.int32),
        pltpu.VMEM((BPW,), jnp.int32),
        pltpu.VMEM((BPW,), jnp.int32),
        pltpu.VMEM((HB, 2 * D), jnp.float32),
        pltpu.VMEM((HB, 2 * D), jnp.float32),
        pltpu.VMEM((HB, 2 * D), jnp.float32),
        pltpu.VMEM((BPW,), jnp.float32),
        pltpu.VMEM((BPW,), jnp.float32),
        pltpu.SemaphoreType.DMA,
        pltpu.SemaphoreType.DMA,
    ],
    compiler_params=pltpu.CompilerParams(needs_layout_passes=False),
)
def _bprmf_sc(user_hbm, itemi_hbm, itemj_hbm, ut_hbm, it_hbm,
              out_i, out_j,
              uix, iix, jix, bu, bi, bj, oi, oj, sem0, sem1):
    wid = lax.axis_index("s") * 2 + lax.axis_index("c")
    base = wid * BPW

    pltpu.sync_copy(user_hbm.at[pl.ds(base, BPW)], uix)
    pltpu.sync_copy(itemi_hbm.at[pl.ds(base, BPW)], iix)
    pltpu.sync_copy(itemj_hbm.at[pl.ds(base, BPW)], jix)

    sems = (sem0, sem1)
    iota = jnp.arange(L, dtype=jnp.int32)

    for side in range(2):
        half = pl.ds(side * D, D)
        sem = sems[side]

        def issue(g, carry):
            off = pl.multiple_of(side * HB + g * L, L)
            uvec = uix[pl.ds(off, L)]
            ivec = iix[pl.ds(off, L)]
            jvec = jix[pl.ds(off, L)]
            for l in range(L):
                row = g * L + l
                pltpu.async_copy(ut_hbm.at[uvec[l]], bu.at[row, half], sem)
                pltpu.async_copy(it_hbm.at[ivec[l]], bi.at[row, half], sem)
                pltpu.async_copy(it_hbm.at[jvec[l]], bj.at[row, half], sem)
            return carry

        lax.fori_loop(0, HG, issue, 0)

    for side in range(2):
        sem = sems[side]

        def drain(g, carry):
            for _ in range(3 * L):
                pltpu.make_async_copy(
                    ut_hbm.at[0], bu.at[0, pl.ds(0, D)], sem
                ).wait()
            return carry

        lax.fori_loop(0, HG, drain, 0)

        def body(g, carry):
            rowids = g * L + iota
            acc_i = jnp.zeros((L,), jnp.float32)
            acc_j = jnp.zeros((L,), jnp.float32)
            for d in range(D):
                colids = jnp.full((L,), side * D + d, dtype=jnp.int32)
                u = plsc.load_gather(bu, [rowids, colids])
                acc_i = acc_i + u * plsc.load_gather(bi, [rowids, colids])
                acc_j = acc_j + u * plsc.load_gather(bj, [rowids, colids])
            off = pl.multiple_of(side * HB + g * L, L)
            oi[pl.ds(off, L)] = acc_i
            oj[pl.ds(off, L)] = acc_j
            return carry

        lax.fori_loop(0, HG, body, 0)

    pltpu.sync_copy(oi, out_i.at[pl.ds(base, BPW)])
    pltpu.sync_copy(oj, out_j.at[pl.ds(base, BPW)])


def kernel(user, item_i, item_j, user_table, item_table):
    return _bprmf_sc(user.astype(jnp.int32), item_i.astype(jnp.int32),
                     item_j.astype(jnp.int32), user_table, item_table)
